# Initial kernel scaffold; baseline (speedup 1.0000x reference)
import jax
import jax.numpy as jnp
from jax import lax
from jax.experimental import pallas as pl
from jax.experimental.pallas import tpu as pltpu

B, D, T, K = 16, 64, 1024, 1024
TC = 512


def _vq_body(x_ref, e_ref, out_ref, loss_ref):
    b = pl.program_id(0)
    t = pl.program_id(1)
    xb = x_ref[0]
    e = e_ref[...]
    scores = lax.dot_general(
        xb, e, (((0,), (0,)), ((), ())), preferred_element_type=jnp.float32
    )
    x_sq = jnp.sum(xb * xb, axis=0)[:, None]
    e_sq = jnp.sum(e * e, axis=0)[None, :]
    d = x_sq - 2.0 * scores + e_sq
    m = jnp.min(d, axis=1, keepdims=True)
    ii = lax.broadcasted_iota(jnp.int32, (TC, K), 1)
    idx = jnp.min(jnp.where(d <= m, ii, K), axis=1)
    onehot = (ii == idx[:, None]).astype(jnp.float32)
    q = lax.dot_general(
        e, onehot, (((1,), (1,)), ((), ())), preferred_element_type=jnp.float32
    )
    out_ref[0] = xb + (q - xb)
    diff = xb - q

    @pl.when(jnp.logical_and(b == 0, t == 0))
    def _():
        loss_ref[0, 0] = 0.0

    loss_ref[0, 0] += jnp.sum(diff * diff)


@jax.jit
def kernel(x_in, e_i_ts):
    grid = (B, T // TC)
    q, loss_sum = pl.pallas_call(
        _vq_body,
        grid=grid,
        in_specs=[
            pl.BlockSpec((1, D, TC), lambda b, t: (b, 0, t)),
            pl.BlockSpec((D, K), lambda b, t: (0, 0)),
        ],
        out_specs=[
            pl.BlockSpec((1, D, TC), lambda b, t: (b, 0, t)),
            pl.BlockSpec((1, 1), lambda b, t: (0, 0)),
        ],
        out_shape=[
            jax.ShapeDtypeStruct((B, D, T), jnp.float32),
            jax.ShapeDtypeStruct((1, 1), jnp.float32),
        ],
        compiler_params=pltpu.CompilerParams(
            dimension_semantics=("arbitrary", "arbitrary"),
        ),
    )(x_in, e_i_ts)
    loss = loss_sum[0, 0] / (B * D * T)
    return (q, loss, loss)


# speedup vs baseline: 1.9707x; 1.9707x over previous
import jax
import jax.numpy as jnp
from jax import lax
from jax.experimental import pallas as pl
from jax.experimental.pallas import tpu as pltpu

B, D, T, K = 16, 64, 1024, 1024
TC = 512


def _vq_body(x_ref, e_ref, out_ref, loss_ref):
    b = pl.program_id(0)
    t = pl.program_id(1)
    xb = x_ref[0]
    e = e_ref[...]
    scores = lax.dot_general(
        xb, e, (((0,), (0,)), ((), ())), preferred_element_type=jnp.float32
    )
    x_sq = jnp.sum(xb * xb, axis=0)[:, None]
    e_sq = jnp.sum(e * e, axis=0)[None, :]
    d = x_sq - 2.0 * scores + e_sq
    m = jnp.min(d, axis=1, keepdims=True)
    ii = lax.broadcasted_iota(jnp.int32, (TC, K), 1)
    idx = jnp.min(jnp.where(d <= m, ii, K), axis=1)
    onehot = (ii == idx[:, None]).astype(jnp.float32)
    q = lax.dot_general(
        e, onehot, (((1,), (1,)), ((), ())), preferred_element_type=jnp.float32
    )
    out_ref[0] = xb + (q - xb)
    diff = xb - q

    @pl.when(jnp.logical_and(b == 0, t == 0))
    def _():
        loss_ref[...] = jnp.zeros((1, 1), jnp.float32)

    loss_ref[...] += jnp.sum(diff * diff).reshape(1, 1)


@jax.jit
def kernel(x_in, e_i_ts):
    grid = (B, T // TC)
    q, loss_sum = pl.pallas_call(
        _vq_body,
        grid=grid,
        in_specs=[
            pl.BlockSpec((1, D, TC), lambda b, t: (b, 0, t)),
            pl.BlockSpec((D, K), lambda b, t: (0, 0)),
        ],
        out_specs=[
            pl.BlockSpec((1, D, TC), lambda b, t: (b, 0, t)),
            pl.BlockSpec((1, 1), lambda b, t: (0, 0)),
        ],
        out_shape=[
            jax.ShapeDtypeStruct((B, D, T), jnp.float32),
            jax.ShapeDtypeStruct((1, 1), jnp.float32),
        ],
        compiler_params=pltpu.CompilerParams(
            dimension_semantics=("arbitrary", "arbitrary"),
        ),
    )(x_in, e_i_ts)
    loss = loss_sum[0, 0] / (B * D * T)
    return (q, loss, loss)
